# baseline (speedup 1.0000x reference)
import functools

import jax
import jax.numpy as jnp
from jax import lax
from jax.experimental import pallas as pl
from jax.experimental.pallas import tpu as pltpu
from jax.experimental.pallas import tpu_sc as plsc

_N = 10000
_E = 320000
_F = 128
_H = 8
_C = 128
_HC = _H * _C
_NCORES = 2
_NSUB = 16
_NW = _NCORES * _NSUB
_EPW = _E // _NW
_G = 16
_NGROUPS = _EPW // _G
_RPS = _N // _NSUB

_f32 = jnp.float32
_i32 = jnp.int32

_SC_MESH = plsc.VectorSubcoreMesh(
    core_axis_name="c", subcore_axis_name="s",
    num_cores=_NCORES, num_subcores=_NSUB)


_BM = 400


def _proj_res_body(x_ref, wl_ref, wr_ref, wlin_ref, blin_ref,
                   xl_ref, xr_ref, res_ref):
    xb = x_ref[...]
    xl_ref[...] = jnp.dot(xb, wl_ref[...], preferred_element_type=_f32)
    xr_ref[...] = jnp.dot(xb, wr_ref[...], preferred_element_type=_f32)
    res_ref[...] = (jnp.dot(xb, wlin_ref[...], preferred_element_type=_f32)
                    + blin_ref[...])


def _proj_res(x, wl, wr, wlin, blin):
    grid = (_N // _BM,)
    return pl.pallas_call(
        _proj_res_body,
        grid=grid,
        in_specs=[
            pl.BlockSpec((_BM, _F), lambda i: (i, 0)),
            pl.BlockSpec((_F, _HC), lambda i: (0, 0)),
            pl.BlockSpec((_F, _HC), lambda i: (0, 0)),
            pl.BlockSpec((_F, _C), lambda i: (0, 0)),
            pl.BlockSpec((1, _C), lambda i: (0, 0)),
        ],
        out_specs=[
            pl.BlockSpec((_BM, _HC), lambda i: (i, 0)),
            pl.BlockSpec((_BM, _HC), lambda i: (i, 0)),
            pl.BlockSpec((_BM, _C), lambda i: (i, 0)),
        ],
        out_shape=[
            jax.ShapeDtypeStruct((_N, _HC), _f32),
            jax.ShapeDtypeStruct((_N, _HC), _f32),
            jax.ShapeDtypeStruct((_N, _C), _f32),
        ],
    )(x, wl, wr, wlin, blin.reshape(1, _C))


def _mid_body(acc_ref, b_ref, wl_ref, wr_ref, xl_ref, xr_ref):
    h = jax.nn.relu(acc_ref[0] + acc_ref[1] + b_ref[...])
    xl_ref[...] = jnp.dot(h, wl_ref[...], preferred_element_type=_f32)
    xr_ref[...] = jnp.dot(h, wr_ref[...], preferred_element_type=_f32)


def _mid(acc_parts, b, wl, wr):
    grid = (_N // _BM,)
    return pl.pallas_call(
        _mid_body,
        grid=grid,
        in_specs=[
            pl.BlockSpec((2, _BM, _C), lambda i: (0, i, 0)),
            pl.BlockSpec((1, _C), lambda i: (0, 0)),
            pl.BlockSpec((_C, _HC), lambda i: (0, 0)),
            pl.BlockSpec((_C, _HC), lambda i: (0, 0)),
        ],
        out_specs=[
            pl.BlockSpec((_BM, _HC), lambda i: (i, 0)),
            pl.BlockSpec((_BM, _HC), lambda i: (i, 0)),
        ],
        out_shape=[
            jax.ShapeDtypeStruct((_N, _HC), _f32),
            jax.ShapeDtypeStruct((_N, _HC), _f32),
        ],
    )(acc_parts, b.reshape(1, _C), wl, wr)


def _final_body(acc_ref, b_ref, res_ref, out_ref):
    out_ref[...] = acc_ref[0] + acc_ref[1] + b_ref[...] + res_ref[...]


def _final(acc_parts, b, res):
    grid = (_N // _BM,)
    return pl.pallas_call(
        _final_body,
        grid=grid,
        in_specs=[
            pl.BlockSpec((2, _BM, _C), lambda i: (0, i, 0)),
            pl.BlockSpec((1, _C), lambda i: (0, 0)),
            pl.BlockSpec((_BM, _C), lambda i: (i, 0)),
        ],
        out_specs=pl.BlockSpec((_BM, _C), lambda i: (i, 0)),
        out_shape=jax.ShapeDtypeStruct((_N, _C), _f32),
    )(acc_parts, b.reshape(1, _C), res)


def _den_sum_body(dp_ref, out_ref):
    out_ref[...] = jnp.sum(dp_ref[...], axis=0) + 1e-30


def _den_sum(den_parts):
    dp = den_parts.reshape(_NW, 625, 128)
    out = pl.pallas_call(
        _den_sum_body,
        grid=(1,),
        in_specs=[pl.BlockSpec((_NW, 625, 128), lambda i: (0, 0, 0))],
        out_specs=pl.BlockSpec((625, 128), lambda i: (0, 0)),
        out_shape=jax.ShapeDtypeStruct((625, 128), _f32),
    )(dp)
    return out.reshape(_N, _H)


def _sc_p1_body(xl_hbm, xr_hbm, src_hbm, dst_hbm, att_hbm, zden_hbm,
                w_hbm, denp_hbm,
                sidx_v, didx_v, xlr_v, xrr_v, att_v, wb_v, den_v,
                sem1, sem2):
    c = lax.axis_index("c")
    s = lax.axis_index("s")
    wid = s * _NCORES + c
    ebase = wid * _EPW
    gwbase = wid * _NGROUPS
    pltpu.sync_copy(att_hbm, att_v)
    pltpu.sync_copy(zden_hbm, den_v)
    lanes = lax.iota(_i32, _G)

    def group(g, carry):
        gbase = ebase + g * _G
        pltpu.sync_copy(src_hbm.at[pl.ds(gbase, _G)], sidx_v)
        pltpu.sync_copy(dst_hbm.at[pl.ds(gbase, _G)], didx_v)
        cp1 = pltpu.async_copy(xl_hbm.at[sidx_v], xlr_v, sem1)
        cp2 = pltpu.async_copy(xr_hbm.at[didx_v], xrr_v, sem2)
        cp1.wait()
        cp2.wait()
        didx = didx_v[...]
        for h in range(_H):
            def jloop(j, acc):
                attj = att_v[pl.ds(h * _C + j * 16, 16)]
                base = j * 16
                for l in range(16):
                    colv = jnp.full((_G,), h * _C + l, _i32) + base
                    a = plsc.load_gather(xlr_v, [lanes, colv])
                    b = plsc.load_gather(xrr_v, [lanes, colv])
                    sv = a + b
                    sv = jnp.maximum(sv, sv * 0.2)
                    acc = acc + sv * attj[l]
                return acc
            logit = lax.fori_loop(0, 8, jloop, jnp.zeros((_G,), _f32))
            wh = jnp.exp(logit)
            wb_v[h, :] = wh
            plsc.addupdate_scatter(den_v, [didx, jnp.full((_G,), h, _i32)], wh)
        pltpu.sync_copy(wb_v, w_hbm.at[gwbase + g])
        return carry

    lax.fori_loop(0, _NGROUPS, group, 0)
    pltpu.sync_copy(den_v, denp_hbm.at[wid])


def _sc_p1(xl, xr, src, dst, att, zden):
    f = pl.kernel(
        _sc_p1_body,
        out_type=[
            jax.ShapeDtypeStruct((_NW * _NGROUPS, _H, _G), _f32),
            jax.ShapeDtypeStruct((_NW, _N, _H), _f32),
        ],
        mesh=_SC_MESH,
        compiler_params=pltpu.CompilerParams(use_tc_tiling_on_sc=False, needs_layout_passes=False),
        scratch_types=[
            pltpu.VMEM((_G,), _i32),
            pltpu.VMEM((_G,), _i32),
            pltpu.VMEM((_G, _HC), _f32),
            pltpu.VMEM((_G, _HC), _f32),
            pltpu.VMEM((_HC,), _f32),
            pltpu.VMEM((_H, _G), _f32),
            pltpu.VMEM((_N, _H), _f32),
            pltpu.SemaphoreType.DMA,
            pltpu.SemaphoreType.DMA,
        ],
    )
    return f(xl, xr, src, dst, att, zden)


def _sc_p2_body(xl_hbm, src_hbm, dst_hbm, w_hbm, den_hbm, zacc_hbm,
                accp_hbm,
                sidx_v, didx_v, xlr_v, wb_v, denr_v, msg_v, acc_sh,
                sem1, sem2):
    c = lax.axis_index("c")
    s = lax.axis_index("s")
    wid = s * _NCORES + c
    ebase = wid * _EPW
    gwbase = wid * _NGROUPS
    pltpu.sync_copy(zacc_hbm, acc_sh.at[pl.ds(s * _RPS, _RPS)])
    plsc.subcore_barrier()
    lanes = lax.iota(_i32, _G)

    def group(g, carry):
        gbase = ebase + g * _G
        pltpu.sync_copy(src_hbm.at[pl.ds(gbase, _G)], sidx_v)
        pltpu.sync_copy(dst_hbm.at[pl.ds(gbase, _G)], didx_v)
        cp1 = pltpu.async_copy(xl_hbm.at[sidx_v], xlr_v, sem1)
        cp2 = pltpu.async_copy(den_hbm.at[didx_v], denr_v, sem2)
        pltpu.sync_copy(w_hbm.at[gwbase + g], wb_v)
        cp2.wait()
        alphas = []
        for h in range(_H):
            dh = plsc.load_gather(denr_v, [lanes, jnp.full((_G,), h, _i32)])
            alphas.append(wb_v[h, :] * 0.125 / dh)
        cp1.wait()

        def chan(cc, carry2):
            m = jnp.zeros((_G,), _f32)
            for h in range(_H):
                colv = jnp.full((_G,), h * _C, _i32) + cc
                m = m + alphas[h] * plsc.load_gather(xlr_v, [lanes, colv])
            plsc.store_scatter(msg_v, [lanes, jnp.full((_G,), 0, _i32) + cc], m)
            return carry2

        lax.fori_loop(0, _C, chan, 0)
        pltpu.sync_copy(msg_v, acc_sh.at[didx_v], add=True)
        return carry

    lax.fori_loop(0, _NGROUPS, group, 0)
    plsc.subcore_barrier()
    pltpu.sync_copy(acc_sh.at[pl.ds(s * _RPS, _RPS)],
                    accp_hbm.at[c, pl.ds(s * _RPS, _RPS)])


def _sc_p2(xl, src, dst, w, den, zacc):
    f = pl.kernel(
        _sc_p2_body,
        out_type=jax.ShapeDtypeStruct((_NCORES, _N, _C), _f32),
        mesh=_SC_MESH,
        compiler_params=pltpu.CompilerParams(use_tc_tiling_on_sc=False, needs_layout_passes=False),
        scratch_types=[
            pltpu.VMEM((_G,), _i32),
            pltpu.VMEM((_G,), _i32),
            pltpu.VMEM((_G, _HC), _f32),
            pltpu.VMEM((_H, _G), _f32),
            pltpu.VMEM((_G, _H), _f32),
            pltpu.VMEM((_G, _C), _f32),
            pltpu.VMEM_SHARED((_N, _C), _f32),
            pltpu.SemaphoreType.DMA,
            pltpu.SemaphoreType.DMA,
        ],
    )
    return f(xl, src, dst, w, den, zacc)


def _gat_layer(xv, src, dst, att, zden, zacc):
    xl, xr = xv
    w, den_parts = _sc_p1(xl, xr, src, dst, att.reshape(_HC), zden)
    den = _den_sum(den_parts.reshape(_NW, _N * _H))
    acc_parts = _sc_p2(xl, src, dst, w, den, zacc)
    return acc_parts


def kernel(x, edge_index, xyz, Wl1, Wr1, att1, b1, Wl2, Wr2, att2, b2,
           Wlin, blin):
    src = edge_index[0]
    dst = edge_index[1]
    zden = jnp.zeros((_N, _H), _f32)
    zacc = jnp.zeros((_RPS, _C), _f32)

    xl1, xr1, res1 = _proj_res(x, Wl1, Wr1, Wlin, blin)
    acc1 = _gat_layer((xl1, xr1), src, dst, att1, zden, zacc)
    xl2, xr2 = _mid(acc1, b1, Wl2, Wr2)
    acc2 = _gat_layer((xl2, xr2), src, dst, att2, zden, zacc)
    return _final(acc2, b2, res1)


# speedup vs baseline: 1.1860x; 1.1860x over previous
import functools

import jax
import jax.numpy as jnp
from jax import lax
from jax.experimental import pallas as pl
from jax.experimental.pallas import tpu as pltpu
from jax.experimental.pallas import tpu_sc as plsc

_N = 10000
_E = 320000
_F = 128
_H = 8
_C = 128
_HC = _H * _C
_NCORES = 2
_NSUB = 16
_NW = _NCORES * _NSUB
_EPW = _E // _NW
_G = 16
_NGROUPS = _EPW // _G
_RPS = _N // _NSUB

_f32 = jnp.float32
_i32 = jnp.int32

_SC_MESH = plsc.VectorSubcoreMesh(
    core_axis_name="c", subcore_axis_name="s",
    num_cores=_NCORES, num_subcores=_NSUB)

_SC_PARAMS = pltpu.CompilerParams(
    use_tc_tiling_on_sc=False, needs_layout_passes=False)


_BM = 400


def _proj_res_body(x_ref, wl_ref, wr_ref, wlin_ref, blin_ref,
                   xl_ref, xr_ref, res_ref):
    xb = x_ref[...]
    xl_ref[...] = jnp.dot(xb, wl_ref[...], preferred_element_type=_f32)
    xr_ref[...] = jnp.dot(xb, wr_ref[...], preferred_element_type=_f32)
    res_ref[...] = (jnp.dot(xb, wlin_ref[...], preferred_element_type=_f32)
                    + blin_ref[...])


def _proj_res(x, wl, wr, wlin, blin):
    grid = (_N // _BM,)
    return pl.pallas_call(
        _proj_res_body,
        grid=grid,
        in_specs=[
            pl.BlockSpec((_BM, _F), lambda i: (i, 0)),
            pl.BlockSpec((_F, _HC), lambda i: (0, 0)),
            pl.BlockSpec((_F, _HC), lambda i: (0, 0)),
            pl.BlockSpec((_F, _C), lambda i: (0, 0)),
            pl.BlockSpec((1, _C), lambda i: (0, 0)),
        ],
        out_specs=[
            pl.BlockSpec((_BM, _HC), lambda i: (i, 0)),
            pl.BlockSpec((_BM, _HC), lambda i: (i, 0)),
            pl.BlockSpec((_BM, _C), lambda i: (i, 0)),
        ],
        out_shape=[
            jax.ShapeDtypeStruct((_N, _HC), _f32),
            jax.ShapeDtypeStruct((_N, _HC), _f32),
            jax.ShapeDtypeStruct((_N, _C), _f32),
        ],
    )(x, wl, wr, wlin, blin.reshape(1, _C))


def _mid_body(acc_ref, b_ref, wl_ref, wr_ref, xl_ref, xr_ref):
    h = jax.nn.relu(acc_ref[0] + acc_ref[1] + b_ref[...])
    xl_ref[...] = jnp.dot(h, wl_ref[...], preferred_element_type=_f32)
    xr_ref[...] = jnp.dot(h, wr_ref[...], preferred_element_type=_f32)


def _mid(acc_parts, b, wl, wr):
    grid = (_N // _BM,)
    return pl.pallas_call(
        _mid_body,
        grid=grid,
        in_specs=[
            pl.BlockSpec((2, _BM, _C), lambda i: (0, i, 0)),
            pl.BlockSpec((1, _C), lambda i: (0, 0)),
            pl.BlockSpec((_C, _HC), lambda i: (0, 0)),
            pl.BlockSpec((_C, _HC), lambda i: (0, 0)),
        ],
        out_specs=[
            pl.BlockSpec((_BM, _HC), lambda i: (i, 0)),
            pl.BlockSpec((_BM, _HC), lambda i: (i, 0)),
        ],
        out_shape=[
            jax.ShapeDtypeStruct((_N, _HC), _f32),
            jax.ShapeDtypeStruct((_N, _HC), _f32),
        ],
    )(acc_parts, b.reshape(1, _C), wl, wr)


def _final_body(acc_ref, b_ref, res_ref, out_ref):
    out_ref[...] = acc_ref[0] + acc_ref[1] + b_ref[...] + res_ref[...]


def _final(acc_parts, b, res):
    grid = (_N // _BM,)
    return pl.pallas_call(
        _final_body,
        grid=grid,
        in_specs=[
            pl.BlockSpec((2, _BM, _C), lambda i: (0, i, 0)),
            pl.BlockSpec((1, _C), lambda i: (0, 0)),
            pl.BlockSpec((_BM, _C), lambda i: (i, 0)),
        ],
        out_specs=pl.BlockSpec((_BM, _C), lambda i: (i, 0)),
        out_shape=jax.ShapeDtypeStruct((_N, _C), _f32),
    )(acc_parts, b.reshape(1, _C), res)


def _den_sum_body(dp_ref, out_ref):
    out_ref[...] = jnp.sum(dp_ref[...], axis=0) + 1e-30


def _den_sum(den_parts):
    dp = den_parts.reshape(_NCORES, 1250, 128)
    out = pl.pallas_call(
        _den_sum_body,
        grid=(1,),
        in_specs=[pl.BlockSpec((_NCORES, 1250, 128), lambda i: (0, 0, 0))],
        out_specs=pl.BlockSpec((1250, 128), lambda i: (0, 0)),
        out_shape=jax.ShapeDtypeStruct((1250, 128), _f32),
    )(dp)
    return out.reshape(_N, 16)


def _sc_p1_body(xl_hbm, xr_hbm, src_hbm, dst_hbm, att_hbm, zden_hbm,
                w_hbm, denp_hbm,
                src_v, dst_v, xlr0, xrr0, xlr1, xrr1, att_v,
                wb0, wb1, dr0, dr1, den_sh,
                semA0, semB0, semA1, semB1, semW0, semW1, semD0, semD1):
    c = lax.axis_index("c")
    s = lax.axis_index("s")
    wid = s * _NCORES + c
    gwbase = wid * _NGROUPS
    pltpu.sync_copy(att_hbm, att_v)
    pltpu.sync_copy(src_hbm.at[wid], src_v)
    pltpu.sync_copy(dst_hbm.at[wid], dst_v)
    pltpu.sync_copy(zden_hbm, den_sh.at[pl.ds(s * _RPS, _RPS)])
    plsc.subcore_barrier()
    lanes = lax.iota(_i32, _G)
    zv = jnp.zeros((_G,), _f32)
    for h in range(_H, 16):
        plsc.store_scatter(dr0, [lanes, jnp.full((_G,), h, _i32)], zv)
        plsc.store_scatter(dr1, [lanes, jnp.full((_G,), h, _i32)], zv)

    def issue(g, xlr, xrr, semA, semB):
        pltpu.async_copy(xl_hbm.at[src_v.at[g]], xlr, semA)
        pltpu.async_copy(xr_hbm.at[dst_v.at[g]], xrr, semB)

    def wait_in(g, xlr, xrr, semA, semB):
        pltpu.make_async_copy(xl_hbm.at[src_v.at[g]], xlr, semA).wait()
        pltpu.make_async_copy(xr_hbm.at[dst_v.at[g]], xrr, semB).wait()

    def compute(g, xlr, xrr, wb, dr, semW, semD):
        @pl.when(g >= 2)
        def _():
            pltpu.make_async_copy(wb, w_hbm.at[gwbase + g - 2], semW).wait()
            pltpu.make_async_copy(dr, den_sh.at[dst_v.at[g - 2]], semD).wait()
        for h in range(_H):
            def jloop(j, acc):
                attj = att_v[pl.ds(h * _C + j * 16, 16)]
                base = j * 16
                for l in range(16):
                    colv = jnp.full((_G,), h * _C + l, _i32) + base
                    a = plsc.load_gather(xlr, [lanes, colv])
                    b = plsc.load_gather(xrr, [lanes, colv])
                    sv = a + b
                    sv = jnp.maximum(sv, sv * 0.2)
                    acc = acc + sv * attj[l]
                return acc
            logit = lax.fori_loop(0, 8, jloop, jnp.zeros((_G,), _f32))
            wh = jnp.exp(logit)
            wb[h, :] = wh
            plsc.store_scatter(dr, [lanes, jnp.full((_G,), h, _i32)], wh)
        pltpu.async_copy(wb, w_hbm.at[gwbase + g], semW)
        pltpu.async_copy(dr, den_sh.at[dst_v.at[g]], semD, add=True)

    issue(0, xlr0, xrr0, semA0, semB0)
    issue(1, xlr1, xrr1, semA1, semB1)

    def pair(k, carry):
        a = k * 2
        b = a + 1
        wait_in(a, xlr0, xrr0, semA0, semB0)
        compute(a, xlr0, xrr0, wb0, dr0, semW0, semD0)
        issue(a + 2, xlr0, xrr0, semA0, semB0)
        wait_in(b, xlr1, xrr1, semA1, semB1)
        compute(b, xlr1, xrr1, wb1, dr1, semW1, semD1)

        @pl.when(b + 2 < _NGROUPS)
        def _():
            issue(b + 2, xlr1, xrr1, semA1, semB1)
        return carry

    lax.fori_loop(0, _NGROUPS // 2, pair, 0)
    gl = _NGROUPS - 1
    wait_in(gl, xlr0, xrr0, semA0, semB0)
    compute(gl, xlr0, xrr0, wb0, dr0, semW0, semD0)
    pltpu.make_async_copy(wb1, w_hbm.at[gwbase + gl - 2], semW1).wait()
    pltpu.make_async_copy(dr1, den_sh.at[dst_v.at[gl - 2]], semD1).wait()
    pltpu.make_async_copy(wb0, w_hbm.at[gwbase + gl], semW0).wait()
    pltpu.make_async_copy(dr0, den_sh.at[dst_v.at[gl]], semD0).wait()
    plsc.subcore_barrier()
    pltpu.sync_copy(den_sh.at[pl.ds(s * _RPS, _RPS)],
                    denp_hbm.at[c, pl.ds(s * _RPS, _RPS)])


def _sc_p1(xl, xr, src, dst, att, zden):
    f = pl.kernel(
        _sc_p1_body,
        out_type=[
            jax.ShapeDtypeStruct((_NW * _NGROUPS, _H, _G), _f32),
            jax.ShapeDtypeStruct((_NCORES, _N, 16), _f32),
        ],
        mesh=_SC_MESH,
        compiler_params=_SC_PARAMS,
        scratch_types=[
            pltpu.VMEM((_NGROUPS, _G), _i32),
            pltpu.VMEM((_NGROUPS, _G), _i32),
            pltpu.VMEM((_G, _HC), _f32),
            pltpu.VMEM((_G, _HC), _f32),
            pltpu.VMEM((_G, _HC), _f32),
            pltpu.VMEM((_G, _HC), _f32),
            pltpu.VMEM((_HC,), _f32),
            pltpu.VMEM((_H, _G), _f32),
            pltpu.VMEM((_H, _G), _f32),
            pltpu.VMEM((_G, 16), _f32),
            pltpu.VMEM((_G, 16), _f32),
            pltpu.VMEM_SHARED((_N, 16), _f32),
            pltpu.SemaphoreType.DMA,
            pltpu.SemaphoreType.DMA,
            pltpu.SemaphoreType.DMA,
            pltpu.SemaphoreType.DMA,
            pltpu.SemaphoreType.DMA,
            pltpu.SemaphoreType.DMA,
            pltpu.SemaphoreType.DMA,
            pltpu.SemaphoreType.DMA,
        ],
    )
    return f(xl, xr, src, dst, att, zden)


_CH = 125
_NCH = _NGROUPS // _CH


def _sc_p2_body(xl_hbm, src_hbm, dst_hbm, w_hbm, den_hbm, zacc_hbm,
                accp_hbm,
                src_v, dst_v, xlr0, xlr1, wb0, wb1, denr0, denr1,
                msg0, msg1, acc_sh,
                semA0, semA1, semW0, semW1, semD0, semD1, semM0, semM1):
    c = lax.axis_index("c")
    s = lax.axis_index("s")
    wid = s * _NCORES + c
    gwbase = wid * _NGROUPS
    pltpu.sync_copy(zacc_hbm, acc_sh.at[pl.ds(s * _RPS, _RPS)])
    plsc.subcore_barrier()
    lanes = lax.iota(_i32, _G)

    def chunk(ci, carry0):
        cbase = ci * _CH

        def issue(g, xlr, wb, denr, semA, semW, semD):
            pltpu.async_copy(xl_hbm.at[src_v.at[g]], xlr, semA)
            pltpu.async_copy(w_hbm.at[gwbase + cbase + g], wb, semW)
            pltpu.async_copy(den_hbm.at[dst_v.at[g]], denr, semD)

        def wait_in(g, xlr, wb, denr, semA, semW, semD):
            pltpu.make_async_copy(xl_hbm.at[src_v.at[g]], xlr, semA).wait()
            pltpu.make_async_copy(
                w_hbm.at[gwbase + cbase + g], wb, semW).wait()
            pltpu.make_async_copy(den_hbm.at[dst_v.at[g]], denr, semD).wait()

        def compute(g, xlr, wb, denr, msg, semM):
            @pl.when(g >= 2)
            def _():
                pltpu.make_async_copy(
                    msg, acc_sh.at[dst_v.at[g - 2]], semM).wait()
            alphas = []
            for h in range(_H):
                dh = plsc.load_gather(denr, [lanes, jnp.full((_G,), h, _i32)])
                alphas.append(wb[h, :] * 0.125 / dh)

            def chan(cc, carry2):
                m = jnp.zeros((_G,), _f32)
                for h in range(_H):
                    colv = jnp.full((_G,), h * _C, _i32) + cc
                    m = m + alphas[h] * plsc.load_gather(xlr, [lanes, colv])
                plsc.store_scatter(
                    msg, [lanes, jnp.full((_G,), 0, _i32) + cc], m)
                return carry2

            lax.fori_loop(0, _C, chan, 0)
            pltpu.async_copy(msg, acc_sh.at[dst_v.at[g]], semM, add=True)

        pltpu.sync_copy(src_hbm.at[wid, pl.ds(cbase, _CH)], src_v)
        pltpu.sync_copy(dst_hbm.at[wid, pl.ds(cbase, _CH)], dst_v)
        issue(0, xlr0, wb0, denr0, semA0, semW0, semD0)
        issue(1, xlr1, wb1, denr1, semA1, semW1, semD1)

        def pair(k, carry):
            a = k * 2
            b = a + 1
            wait_in(a, xlr0, wb0, denr0, semA0, semW0, semD0)
            compute(a, xlr0, wb0, denr0, msg0, semM0)
            issue(a + 2, xlr0, wb0, denr0, semA0, semW0, semD0)
            wait_in(b, xlr1, wb1, denr1, semA1, semW1, semD1)
            compute(b, xlr1, wb1, denr1, msg1, semM1)

            @pl.when(b + 2 < _CH)
            def _():
                issue(b + 2, xlr1, wb1, denr1, semA1, semW1, semD1)
            return carry

        lax.fori_loop(0, _CH // 2, pair, 0)
        gl = _CH - 1
        wait_in(gl, xlr0, wb0, denr0, semA0, semW0, semD0)
        compute(gl, xlr0, wb0, denr0, msg0, semM0)
        pltpu.make_async_copy(msg1, acc_sh.at[dst_v.at[gl - 2]], semM1).wait()
        pltpu.make_async_copy(msg0, acc_sh.at[dst_v.at[gl]], semM0).wait()
        return carry0

    lax.fori_loop(0, _NCH, chunk, 0)
    plsc.subcore_barrier()
    pltpu.sync_copy(acc_sh.at[pl.ds(s * _RPS, _RPS)],
                    accp_hbm.at[c, pl.ds(s * _RPS, _RPS)])


def _sc_p2(xl, src, dst, w, den, zacc):
    f = pl.kernel(
        _sc_p2_body,
        out_type=jax.ShapeDtypeStruct((_NCORES, _N, _C), _f32),
        mesh=_SC_MESH,
        compiler_params=_SC_PARAMS,
        scratch_types=[
            pltpu.VMEM((_CH, _G), _i32),
            pltpu.VMEM((_CH, _G), _i32),
            pltpu.VMEM((_G, _HC), _f32),
            pltpu.VMEM((_G, _HC), _f32),
            pltpu.VMEM((_H, _G), _f32),
            pltpu.VMEM((_H, _G), _f32),
            pltpu.VMEM((_G, 16), _f32),
            pltpu.VMEM((_G, 16), _f32),
            pltpu.VMEM((_G, _C), _f32),
            pltpu.VMEM((_G, _C), _f32),
            pltpu.VMEM_SHARED((_N, _C), _f32),
            pltpu.SemaphoreType.DMA,
            pltpu.SemaphoreType.DMA,
            pltpu.SemaphoreType.DMA,
            pltpu.SemaphoreType.DMA,
            pltpu.SemaphoreType.DMA,
            pltpu.SemaphoreType.DMA,
            pltpu.SemaphoreType.DMA,
            pltpu.SemaphoreType.DMA,
        ],
    )
    return f(xl, src, dst, w, den, zacc)


def _gat_layer(xv, src, dst, att, zden, zacc):
    xl, xr = xv
    w, den_parts = _sc_p1(xl, xr, src, dst, att.reshape(_HC), zden)
    den = _den_sum(den_parts.reshape(_NCORES, _N * 16))
    acc_parts = _sc_p2(xl, src, dst, w, den, zacc)
    return acc_parts


def kernel(x, edge_index, xyz, Wl1, Wr1, att1, b1, Wl2, Wr2, att2, b2,
           Wlin, blin):
    src = edge_index[0].reshape(_NW, _NGROUPS, _G)
    dst = edge_index[1].reshape(_NW, _NGROUPS, _G)
    zden = jnp.zeros((_RPS, 16), _f32)
    zacc = jnp.zeros((_RPS, _C), _f32)

    xl1, xr1, res1 = _proj_res(x, Wl1, Wr1, Wlin, blin)
    acc1 = _gat_layer((xl1, xr1), src, dst, att1, zden, zacc)
    xl2, xr2 = _mid(acc1, b1, Wl2, Wr2)
    acc2 = _gat_layer((xl2, xr2), src, dst, att2, zden, zacc)
    return _final(acc2, b2, res1)


# speedup vs baseline: 7.9503x; 6.7035x over previous
import functools

import jax
import jax.numpy as jnp
from jax import lax
from jax.experimental import pallas as pl
from jax.experimental.pallas import tpu as pltpu
from jax.experimental.pallas import tpu_sc as plsc

_N = 10000
_E = 320000
_F = 128
_H = 8
_C = 128
_HC = _H * _C
_NCORES = 2
_NSUB = 16
_NW = _NCORES * _NSUB
_EPW = _E // _NW
_G = 16
_NGROUPS = _EPW // _G
_RPS = _N // _NSUB

_f32 = jnp.float32
_i32 = jnp.int32

_SC_MESH = plsc.VectorSubcoreMesh(
    core_axis_name="c", subcore_axis_name="s",
    num_cores=_NCORES, num_subcores=_NSUB)

_SC_PARAMS = pltpu.CompilerParams(
    use_tc_tiling_on_sc=False, needs_layout_passes=False)


_BM = 400


def _proj_res_body(x_ref, wl_ref, wr_ref, wlin_ref, blin_ref,
                   xl_ref, xr_ref, res_ref):
    xb = x_ref[...]
    xl_ref[...] = jnp.dot(xb, wl_ref[...], preferred_element_type=_f32)
    xr_ref[...] = jnp.dot(xb, wr_ref[...], preferred_element_type=_f32)
    res_ref[...] = (jnp.dot(xb, wlin_ref[...], preferred_element_type=_f32)
                    + blin_ref[...])


def _proj_res(x, wl, wr, wlin, blin):
    grid = (_N // _BM,)
    return pl.pallas_call(
        _proj_res_body,
        grid=grid,
        in_specs=[
            pl.BlockSpec((_BM, _F), lambda i: (i, 0)),
            pl.BlockSpec((_F, _HC), lambda i: (0, 0)),
            pl.BlockSpec((_F, _HC), lambda i: (0, 0)),
            pl.BlockSpec((_F, _C), lambda i: (0, 0)),
            pl.BlockSpec((1, _C), lambda i: (0, 0)),
        ],
        out_specs=[
            pl.BlockSpec((_BM, _HC), lambda i: (i, 0)),
            pl.BlockSpec((_BM, _HC), lambda i: (i, 0)),
            pl.BlockSpec((_BM, _C), lambda i: (i, 0)),
        ],
        out_shape=[
            jax.ShapeDtypeStruct((_N, _HC), _f32),
            jax.ShapeDtypeStruct((_N, _HC), _f32),
            jax.ShapeDtypeStruct((_N, _C), _f32),
        ],
    )(x, wl, wr, wlin, blin.reshape(1, _C))


def _mid_body(acc_ref, b_ref, wl_ref, wr_ref, xl_ref, xr_ref):
    h = jax.nn.relu(acc_ref[0] + acc_ref[1] + b_ref[...])
    xl_ref[...] = jnp.dot(h, wl_ref[...], preferred_element_type=_f32)
    xr_ref[...] = jnp.dot(h, wr_ref[...], preferred_element_type=_f32)


def _mid(acc_parts, b, wl, wr):
    grid = (_N // _BM,)
    return pl.pallas_call(
        _mid_body,
        grid=grid,
        in_specs=[
            pl.BlockSpec((2, _BM, _C), lambda i: (0, i, 0)),
            pl.BlockSpec((1, _C), lambda i: (0, 0)),
            pl.BlockSpec((_C, _HC), lambda i: (0, 0)),
            pl.BlockSpec((_C, _HC), lambda i: (0, 0)),
        ],
        out_specs=[
            pl.BlockSpec((_BM, _HC), lambda i: (i, 0)),
            pl.BlockSpec((_BM, _HC), lambda i: (i, 0)),
        ],
        out_shape=[
            jax.ShapeDtypeStruct((_N, _HC), _f32),
            jax.ShapeDtypeStruct((_N, _HC), _f32),
        ],
    )(acc_parts, b.reshape(1, _C), wl, wr)


def _final_body(acc_ref, b_ref, res_ref, out_ref):
    out_ref[...] = acc_ref[0] + acc_ref[1] + b_ref[...] + res_ref[...]


def _final(acc_parts, b, res):
    grid = (_N // _BM,)
    return pl.pallas_call(
        _final_body,
        grid=grid,
        in_specs=[
            pl.BlockSpec((2, _BM, _C), lambda i: (0, i, 0)),
            pl.BlockSpec((1, _C), lambda i: (0, 0)),
            pl.BlockSpec((_BM, _C), lambda i: (i, 0)),
        ],
        out_specs=pl.BlockSpec((_BM, _C), lambda i: (i, 0)),
        out_shape=jax.ShapeDtypeStruct((_N, _C), _f32),
    )(acc_parts, b.reshape(1, _C), res)


def _den_sum_body(dp_ref, out_ref):
    out_ref[...] = jnp.sum(dp_ref[...], axis=0) + 1e-30


def _den_sum(den_parts):
    dp = den_parts.reshape(_NCORES, 1250, 128)
    out = pl.pallas_call(
        _den_sum_body,
        grid=(1,),
        in_specs=[pl.BlockSpec((_NCORES, 1250, 128), lambda i: (0, 0, 0))],
        out_specs=pl.BlockSpec((1250, 128), lambda i: (0, 0)),
        out_shape=jax.ShapeDtypeStruct((1250, 128), _f32),
    )(dp)
    return out.reshape(_N, 16)


def _sc_p1_body(xl_hbm, xr_hbm, src_hbm, dst_hbm, att_hbm, zden_hbm,
                w_hbm, denp_hbm,
                src_v, dst_v, xlr0, xrr0, xlr1, xrr1, att_v,
                wb0, wb1, dr0, dr1, den_sh,
                semA0, semB0, semA1, semB1, semW0, semW1, semD0, semD1):
    c = lax.axis_index("c")
    s = lax.axis_index("s")
    wid = s * _NCORES + c
    gwbase = wid * _NGROUPS
    pltpu.sync_copy(att_hbm, att_v)
    pltpu.sync_copy(src_hbm.at[wid], src_v)
    pltpu.sync_copy(dst_hbm.at[wid], dst_v)
    pltpu.sync_copy(zden_hbm, den_sh.at[pl.ds(s * _RPS, _RPS)])
    plsc.subcore_barrier()
    lanes = lax.iota(_i32, _G)
    zv = jnp.zeros((_G,), _f32)
    for h in range(_H, 16):
        plsc.store_scatter(dr0, [lanes, jnp.full((_G,), h, _i32)], zv)
        plsc.store_scatter(dr1, [lanes, jnp.full((_G,), h, _i32)], zv)

    def issue(g, xlr, xrr, semA, semB):
        pltpu.async_copy(xl_hbm.at[src_v.at[g]], xlr, semA)
        pltpu.async_copy(xr_hbm.at[dst_v.at[g]], xrr, semB)

    def wait_in(g, xlr, xrr, semA, semB):
        pltpu.make_async_copy(xl_hbm.at[src_v.at[g]], xlr, semA).wait()
        pltpu.make_async_copy(xr_hbm.at[dst_v.at[g]], xrr, semB).wait()

    def compute(g, xlr, xrr, wb, dr, semW, semD):
        @pl.when(g >= 2)
        def _():
            pltpu.make_async_copy(wb, w_hbm.at[gwbase + g - 2], semW).wait()
            pltpu.make_async_copy(dr, den_sh.at[dst_v.at[g - 2]], semD).wait()
        for h in range(_H):
            hbase = h * _C

            def jloop(j, accs):
                col = pl.ds(hbase + j * 16, 16)
                attj = att_v[pl.ds(hbase + j * 16, 16)]
                out = []
                for e in range(_G):
                    sv = xlr[e, col] + xrr[e, col]
                    sv = jnp.maximum(sv, sv * 0.2)
                    out.append(accs[e] + sv * attj)
                return tuple(out)

            accs = lax.fori_loop(
                0, 8, jloop,
                tuple(jnp.zeros((_G,), _f32) for _ in range(_G)))
            wv = jnp.zeros((_G,), _f32)
            for e in range(_G):
                wv = jnp.where(lanes == e, jnp.sum(accs[e]), wv)
            wh = jnp.exp(wv)
            wb[h, :] = wh
            plsc.store_scatter(dr, [lanes, jnp.full((_G,), h, _i32)], wh)
        pltpu.async_copy(wb, w_hbm.at[gwbase + g], semW)
        pltpu.async_copy(dr, den_sh.at[dst_v.at[g]], semD, add=True)

    issue(0, xlr0, xrr0, semA0, semB0)
    issue(1, xlr1, xrr1, semA1, semB1)

    def pair(k, carry):
        a = k * 2
        b = a + 1
        wait_in(a, xlr0, xrr0, semA0, semB0)
        compute(a, xlr0, xrr0, wb0, dr0, semW0, semD0)

        @pl.when(a + 2 < _NGROUPS)
        def _():
            issue(a + 2, xlr0, xrr0, semA0, semB0)

        @pl.when(b < _NGROUPS)
        def _():
            wait_in(b, xlr1, xrr1, semA1, semB1)
            compute(b, xlr1, xrr1, wb1, dr1, semW1, semD1)

            @pl.when(b + 2 < _NGROUPS)
            def _():
                issue(b + 2, xlr1, xrr1, semA1, semB1)
        return carry

    lax.fori_loop(0, (_NGROUPS + 1) // 2, pair, 0)
    pltpu.make_async_copy(
        wb0, w_hbm.at[gwbase + _NGROUPS - 1], semW0).wait()
    pltpu.make_async_copy(
        dr0, den_sh.at[dst_v.at[_NGROUPS - 1]], semD0).wait()
    pltpu.make_async_copy(
        wb1, w_hbm.at[gwbase + _NGROUPS - 2], semW1).wait()
    pltpu.make_async_copy(
        dr1, den_sh.at[dst_v.at[_NGROUPS - 2]], semD1).wait()
    plsc.subcore_barrier()
    pltpu.sync_copy(den_sh.at[pl.ds(s * _RPS, _RPS)],
                    denp_hbm.at[c, pl.ds(s * _RPS, _RPS)])


def _sc_p1(xl, xr, src, dst, att, zden):
    f = pl.kernel(
        _sc_p1_body,
        out_type=[
            jax.ShapeDtypeStruct((_NW * _NGROUPS, _H, _G), _f32),
            jax.ShapeDtypeStruct((_NCORES, _N, 16), _f32),
        ],
        mesh=_SC_MESH,
        compiler_params=_SC_PARAMS,
        scratch_types=[
            pltpu.VMEM((_NGROUPS, _G), _i32),
            pltpu.VMEM((_NGROUPS, _G), _i32),
            pltpu.VMEM((_G, _HC), _f32),
            pltpu.VMEM((_G, _HC), _f32),
            pltpu.VMEM((_G, _HC), _f32),
            pltpu.VMEM((_G, _HC), _f32),
            pltpu.VMEM((_HC,), _f32),
            pltpu.VMEM((_H, _G), _f32),
            pltpu.VMEM((_H, _G), _f32),
            pltpu.VMEM((_G, 16), _f32),
            pltpu.VMEM((_G, 16), _f32),
            pltpu.VMEM_SHARED((_N, 16), _f32),
            pltpu.SemaphoreType.DMA,
            pltpu.SemaphoreType.DMA,
            pltpu.SemaphoreType.DMA,
            pltpu.SemaphoreType.DMA,
            pltpu.SemaphoreType.DMA,
            pltpu.SemaphoreType.DMA,
            pltpu.SemaphoreType.DMA,
            pltpu.SemaphoreType.DMA,
        ],
    )
    return f(xl, xr, src, dst, att, zden)


_CH = 125
_NCH = _NGROUPS // _CH


def _sc_p2_body(xl_hbm, src_hbm, dst_hbm, w_hbm, den_hbm, zacc_hbm,
                accp_hbm,
                src_v, dst_v, xlr0, xlr1, wb0, wb1, denr0, denr1,
                msg0, msg1, acc_sh,
                semA0, semA1, semW0, semW1, semD0, semD1, semM0, semM1):
    c = lax.axis_index("c")
    s = lax.axis_index("s")
    wid = s * _NCORES + c
    gwbase = wid * _NGROUPS
    pltpu.sync_copy(zacc_hbm, acc_sh.at[pl.ds(s * _RPS, _RPS)])
    plsc.subcore_barrier()
    lanes = lax.iota(_i32, _G)

    def chunk(ci, carry0):
        cbase = ci * _CH

        def issue(g, xlr, wb, denr, semA, semW, semD):
            pltpu.async_copy(xl_hbm.at[src_v.at[g]], xlr, semA)
            pltpu.async_copy(w_hbm.at[gwbase + cbase + g], wb, semW)
            pltpu.async_copy(den_hbm.at[dst_v.at[g]], denr, semD)

        def wait_in(g, xlr, wb, denr, semA, semW, semD):
            pltpu.make_async_copy(xl_hbm.at[src_v.at[g]], xlr, semA).wait()
            pltpu.make_async_copy(
                w_hbm.at[gwbase + cbase + g], wb, semW).wait()
            pltpu.make_async_copy(den_hbm.at[dst_v.at[g]], denr, semD).wait()

        def compute(g, xlr, wb, denr, msg, semM):
            @pl.when(g >= 2)
            def _():
                pltpu.make_async_copy(
                    msg, acc_sh.at[dst_v.at[g - 2]], semM).wait()
            alpha_sc = []
            for h in range(_H):
                dh = plsc.load_gather(denr, [lanes, jnp.full((_G,), h, _i32)])
                ah = wb[h, :] * 0.125 / dh
                alpha_sc.append([ah[e] for e in range(_G)])

            def jloop(j, carry2):
                for e in range(_G):
                    m = jnp.zeros((16,), _f32)
                    for h in range(_H):
                        m = m + alpha_sc[h][e] * xlr[e, pl.ds(h * _C + j * 16, 16)]
                    msg[e, pl.ds(j * 16, 16)] = m
                return carry2

            lax.fori_loop(0, 8, jloop, 0)
            pltpu.async_copy(msg, acc_sh.at[dst_v.at[g]], semM, add=True)

        pltpu.sync_copy(src_hbm.at[wid, pl.ds(cbase, _CH)], src_v)
        pltpu.sync_copy(dst_hbm.at[wid, pl.ds(cbase, _CH)], dst_v)
        issue(0, xlr0, wb0, denr0, semA0, semW0, semD0)
        issue(1, xlr1, wb1, denr1, semA1, semW1, semD1)

        def pair(k, carry):
            a = k * 2
            b = a + 1
            wait_in(a, xlr0, wb0, denr0, semA0, semW0, semD0)
            compute(a, xlr0, wb0, denr0, msg0, semM0)

            @pl.when(a + 2 < _CH)
            def _():
                issue(a + 2, xlr0, wb0, denr0, semA0, semW0, semD0)

            @pl.when(b < _CH)
            def _():
                wait_in(b, xlr1, wb1, denr1, semA1, semW1, semD1)
                compute(b, xlr1, wb1, denr1, msg1, semM1)

                @pl.when(b + 2 < _CH)
                def _():
                    issue(b + 2, xlr1, wb1, denr1, semA1, semW1, semD1)
            return carry

        lax.fori_loop(0, (_CH + 1) // 2, pair, 0)
        pltpu.make_async_copy(msg0, acc_sh.at[dst_v.at[_CH - 1]], semM0).wait()
        pltpu.make_async_copy(msg1, acc_sh.at[dst_v.at[_CH - 2]], semM1).wait()
        return carry0

    lax.fori_loop(0, _NCH, chunk, 0)
    plsc.subcore_barrier()
    pltpu.sync_copy(acc_sh.at[pl.ds(s * _RPS, _RPS)],
                    accp_hbm.at[c, pl.ds(s * _RPS, _RPS)])


def _sc_p2(xl, src, dst, w, den, zacc):
    f = pl.kernel(
        _sc_p2_body,
        out_type=jax.ShapeDtypeStruct((_NCORES, _N, _C), _f32),
        mesh=_SC_MESH,
        compiler_params=_SC_PARAMS,
        scratch_types=[
            pltpu.VMEM((_CH, _G), _i32),
            pltpu.VMEM((_CH, _G), _i32),
            pltpu.VMEM((_G, _HC), _f32),
            pltpu.VMEM((_G, _HC), _f32),
            pltpu.VMEM((_H, _G), _f32),
            pltpu.VMEM((_H, _G), _f32),
            pltpu.VMEM((_G, 16), _f32),
            pltpu.VMEM((_G, 16), _f32),
            pltpu.VMEM((_G, _C), _f32),
            pltpu.VMEM((_G, _C), _f32),
            pltpu.VMEM_SHARED((_N, _C), _f32),
            pltpu.SemaphoreType.DMA,
            pltpu.SemaphoreType.DMA,
            pltpu.SemaphoreType.DMA,
            pltpu.SemaphoreType.DMA,
            pltpu.SemaphoreType.DMA,
            pltpu.SemaphoreType.DMA,
            pltpu.SemaphoreType.DMA,
            pltpu.SemaphoreType.DMA,
        ],
    )
    return f(xl, src, dst, w, den, zacc)


def _gat_layer(xv, src, dst, att, zden, zacc):
    xl, xr = xv
    w, den_parts = _sc_p1(xl, xr, src, dst, att.reshape(_HC), zden)
    den = _den_sum(den_parts.reshape(_NCORES, _N * 16))
    acc_parts = _sc_p2(xl, src, dst, w, den, zacc)
    return acc_parts


def kernel(x, edge_index, xyz, Wl1, Wr1, att1, b1, Wl2, Wr2, att2, b2,
           Wlin, blin):
    src = edge_index[0].reshape(_NW, _NGROUPS, _G)
    dst = edge_index[1].reshape(_NW, _NGROUPS, _G)
    zden = jnp.zeros((_RPS, 16), _f32)
    zacc = jnp.zeros((_RPS, _C), _f32)

    xl1, xr1, res1 = _proj_res(x, Wl1, Wr1, Wlin, blin)
    acc1 = _gat_layer((xl1, xr1), src, dst, att1, zden, zacc)
    xl2, xr2 = _mid(acc1, b1, Wl2, Wr2)
    acc2 = _gat_layer((xl2, xr2), src, dst, att2, zden, zacc)
    return _final(acc2, b2, res1)


# speedup vs baseline: 8.3500x; 1.0503x over previous
import functools

import jax
import jax.numpy as jnp
import numpy as np
from jax import lax
from jax.experimental import pallas as pl
from jax.experimental.pallas import tpu as pltpu
from jax.experimental.pallas import tpu_sc as plsc

_N = 10000
_E = 320000
_F = 128
_H = 8
_C = 128
_HC = _H * _C
_NCORES = 2
_NSUB = 16
_NW = _NCORES * _NSUB
_EPW = _E // _NW
_G = 16
_NGROUPS = _EPW // _G
_RPS = _N // _NSUB

_f32 = jnp.float32
_i32 = jnp.int32

_SC_MESH = plsc.VectorSubcoreMesh(
    core_axis_name="c", subcore_axis_name="s",
    num_cores=_NCORES, num_subcores=_NSUB)

_SC_PARAMS = pltpu.CompilerParams(
    use_tc_tiling_on_sc=False, needs_layout_passes=False)

_bf16 = jnp.bfloat16


def _mk_colperm():
    tp = np.empty(128, np.int64)
    for j in range(4):
        for k in range(16):
            tp[32 * j + 2 * k] = 32 * j + k
            tp[32 * j + 2 * k + 1] = 32 * j + 16 + k
    return np.concatenate([h * 128 + tp for h in range(_H)])


_COLPERM = _mk_colperm()


_BM = 400


def _proj_res_body(x_ref, wl_ref, wr_ref, wlin_ref, blin_ref,
                   xl_ref, xr_ref, res_ref):
    xb = x_ref[...]
    xl_ref[...] = jnp.dot(
        xb, wl_ref[...], preferred_element_type=_f32).astype(_bf16)
    xr_ref[...] = jnp.dot(
        xb, wr_ref[...], preferred_element_type=_f32).astype(_bf16)
    res_ref[...] = (jnp.dot(xb, wlin_ref[...], preferred_element_type=_f32)
                    + blin_ref[...])


def _proj_res(x, wl, wr, wlin, blin):
    grid = (_N // _BM,)
    return pl.pallas_call(
        _proj_res_body,
        grid=grid,
        in_specs=[
            pl.BlockSpec((_BM, _F), lambda i: (i, 0)),
            pl.BlockSpec((_F, _HC), lambda i: (0, 0)),
            pl.BlockSpec((_F, _HC), lambda i: (0, 0)),
            pl.BlockSpec((_F, _C), lambda i: (0, 0)),
            pl.BlockSpec((1, _C), lambda i: (0, 0)),
        ],
        out_specs=[
            pl.BlockSpec((_BM, _HC), lambda i: (i, 0)),
            pl.BlockSpec((_BM, _HC), lambda i: (i, 0)),
            pl.BlockSpec((_BM, _C), lambda i: (i, 0)),
        ],
        out_shape=[
            jax.ShapeDtypeStruct((_N, _HC), _bf16),
            jax.ShapeDtypeStruct((_N, _HC), _bf16),
            jax.ShapeDtypeStruct((_N, _C), _f32),
        ],
    )(x, wl, wr, wlin, blin.reshape(1, _C))


def _mid_body(acc_ref, b_ref, wl_ref, wr_ref, xl_ref, xr_ref):
    h = jax.nn.relu(acc_ref[0] + acc_ref[1] + b_ref[...])
    xl_ref[...] = jnp.dot(
        h, wl_ref[...], preferred_element_type=_f32).astype(_bf16)
    xr_ref[...] = jnp.dot(
        h, wr_ref[...], preferred_element_type=_f32).astype(_bf16)


def _mid(acc_parts, b, wl, wr):
    grid = (_N // _BM,)
    return pl.pallas_call(
        _mid_body,
        grid=grid,
        in_specs=[
            pl.BlockSpec((2, _BM, _C), lambda i: (0, i, 0)),
            pl.BlockSpec((1, _C), lambda i: (0, 0)),
            pl.BlockSpec((_C, _HC), lambda i: (0, 0)),
            pl.BlockSpec((_C, _HC), lambda i: (0, 0)),
        ],
        out_specs=[
            pl.BlockSpec((_BM, _HC), lambda i: (i, 0)),
            pl.BlockSpec((_BM, _HC), lambda i: (i, 0)),
        ],
        out_shape=[
            jax.ShapeDtypeStruct((_N, _HC), _bf16),
            jax.ShapeDtypeStruct((_N, _HC), _bf16),
        ],
    )(acc_parts, b.reshape(1, _C), wl, wr)


def _final_body(acc_ref, b_ref, res_ref, out_ref):
    out_ref[...] = acc_ref[0] + acc_ref[1] + b_ref[...] + res_ref[...]


def _final(acc_parts, b, res):
    grid = (_N // _BM,)
    return pl.pallas_call(
        _final_body,
        grid=grid,
        in_specs=[
            pl.BlockSpec((2, _BM, _C), lambda i: (0, i, 0)),
            pl.BlockSpec((1, _C), lambda i: (0, 0)),
            pl.BlockSpec((_BM, _C), lambda i: (i, 0)),
        ],
        out_specs=pl.BlockSpec((_BM, _C), lambda i: (i, 0)),
        out_shape=jax.ShapeDtypeStruct((_N, _C), _f32),
    )(acc_parts, b.reshape(1, _C), res)


def _den_sum_body(dp_ref, out_ref):
    out_ref[...] = jnp.sum(dp_ref[...], axis=0) + 1e-30


def _den_sum(den_parts):
    dp = den_parts.reshape(_NCORES, 1250, 128)
    out = pl.pallas_call(
        _den_sum_body,
        grid=(1,),
        in_specs=[pl.BlockSpec((_NCORES, 1250, 128), lambda i: (0, 0, 0))],
        out_specs=pl.BlockSpec((1250, 128), lambda i: (0, 0)),
        out_shape=jax.ShapeDtypeStruct((1250, 128), _f32),
    )(dp)
    return out.reshape(_N, 16)


def _sc_p1_body(xl_hbm, xr_hbm, src_hbm, dst_hbm, att_hbm, zden_hbm,
                w_hbm, denp_hbm,
                src_v, dst_v, xlr0, xrr0, xlr1, xrr1, att_v,
                wb0, wb1, dr0, dr1, den_sh,
                semA0, semB0, semA1, semB1, semW0, semW1, semD0, semD1):
    c = lax.axis_index("c")
    s = lax.axis_index("s")
    wid = s * _NCORES + c
    gwbase = wid * _NGROUPS
    pltpu.sync_copy(att_hbm, att_v)
    pltpu.sync_copy(src_hbm.at[wid], src_v)
    pltpu.sync_copy(dst_hbm.at[wid], dst_v)
    pltpu.sync_copy(zden_hbm, den_sh.at[pl.ds(s * _RPS, _RPS)])
    plsc.subcore_barrier()
    lanes = lax.iota(_i32, _G)
    zv = jnp.zeros((_G,), _f32)
    for h in range(_H, 16):
        plsc.store_scatter(dr0, [lanes, jnp.full((_G,), h, _i32)], zv)
        plsc.store_scatter(dr1, [lanes, jnp.full((_G,), h, _i32)], zv)

    def issue(g, xlr, xrr, semA, semB):
        pltpu.async_copy(xl_hbm.at[src_v.at[g]], xlr, semA)
        pltpu.async_copy(xr_hbm.at[dst_v.at[g]], xrr, semB)

    def wait_in(g, xlr, xrr, semA, semB):
        pltpu.make_async_copy(xl_hbm.at[src_v.at[g]], xlr, semA).wait()
        pltpu.make_async_copy(xr_hbm.at[dst_v.at[g]], xrr, semB).wait()

    def compute(g, xlr, xrr, wb, dr, semW, semD):
        @pl.when(g >= 2)
        def _():
            pltpu.make_async_copy(wb, w_hbm.at[gwbase + g - 2], semW).wait()
            pltpu.make_async_copy(dr, den_sh.at[dst_v.at[g - 2]], semD).wait()
        for h in range(_H):
            hbase = h * _C

            def jloop(j, accs):
                col = pl.ds(hbase + j * 32, 32)
                attlo = att_v[pl.ds(hbase + j * 32, 16)]
                atthi = att_v[pl.ds(hbase + j * 32 + 16, 16)]
                out = []
                for e in range(_G):
                    sv = xlr[e, col] + xrr[e, col]
                    sv = jnp.maximum(sv, sv * jnp.asarray(0.2, _bf16))
                    lo, hi = plsc.unpack(
                        sv, format=plsc.PackFormat.INTERLEAVED)
                    out.append(accs[e] + lo * attlo + hi * atthi)
                return tuple(out)

            accs = lax.fori_loop(
                0, 4, jloop,
                tuple(jnp.zeros((_G,), _f32) for _ in range(_G)))
            wv = jnp.zeros((_G,), _f32)
            for e in range(_G):
                wv = jnp.where(lanes == e, jnp.sum(accs[e]), wv)
            wh = jnp.exp(wv)
            wb[h, :] = wh
            plsc.store_scatter(dr, [lanes, jnp.full((_G,), h, _i32)], wh)
        pltpu.async_copy(wb, w_hbm.at[gwbase + g], semW)
        pltpu.async_copy(dr, den_sh.at[dst_v.at[g]], semD, add=True)

    issue(0, xlr0, xrr0, semA0, semB0)
    issue(1, xlr1, xrr1, semA1, semB1)

    def pair(k, carry):
        a = k * 2
        b = a + 1
        wait_in(a, xlr0, xrr0, semA0, semB0)
        compute(a, xlr0, xrr0, wb0, dr0, semW0, semD0)

        @pl.when(a + 2 < _NGROUPS)
        def _():
            issue(a + 2, xlr0, xrr0, semA0, semB0)

        @pl.when(b < _NGROUPS)
        def _():
            wait_in(b, xlr1, xrr1, semA1, semB1)
            compute(b, xlr1, xrr1, wb1, dr1, semW1, semD1)

            @pl.when(b + 2 < _NGROUPS)
            def _():
                issue(b + 2, xlr1, xrr1, semA1, semB1)
        return carry

    lax.fori_loop(0, (_NGROUPS + 1) // 2, pair, 0)
    pltpu.make_async_copy(
        wb0, w_hbm.at[gwbase + _NGROUPS - 1], semW0).wait()
    pltpu.make_async_copy(
        dr0, den_sh.at[dst_v.at[_NGROUPS - 1]], semD0).wait()
    pltpu.make_async_copy(
        wb1, w_hbm.at[gwbase + _NGROUPS - 2], semW1).wait()
    pltpu.make_async_copy(
        dr1, den_sh.at[dst_v.at[_NGROUPS - 2]], semD1).wait()
    plsc.subcore_barrier()
    pltpu.sync_copy(den_sh.at[pl.ds(s * _RPS, _RPS)],
                    denp_hbm.at[c, pl.ds(s * _RPS, _RPS)])


def _sc_p1(xl, xr, src, dst, att, zden):
    f = pl.kernel(
        _sc_p1_body,
        out_type=[
            jax.ShapeDtypeStruct((_NW * _NGROUPS, _H, _G), _f32),
            jax.ShapeDtypeStruct((_NCORES, _N, 16), _f32),
        ],
        mesh=_SC_MESH,
        compiler_params=_SC_PARAMS,
        scratch_types=[
            pltpu.VMEM((_NGROUPS, _G), _i32),
            pltpu.VMEM((_NGROUPS, _G), _i32),
            pltpu.VMEM((_G, _HC), _bf16),
            pltpu.VMEM((_G, _HC), _bf16),
            pltpu.VMEM((_G, _HC), _bf16),
            pltpu.VMEM((_G, _HC), _bf16),
            pltpu.VMEM((_HC,), _f32),
            pltpu.VMEM((_H, _G), _f32),
            pltpu.VMEM((_H, _G), _f32),
            pltpu.VMEM((_G, 16), _f32),
            pltpu.VMEM((_G, 16), _f32),
            pltpu.VMEM_SHARED((_N, 16), _f32),
            pltpu.SemaphoreType.DMA,
            pltpu.SemaphoreType.DMA,
            pltpu.SemaphoreType.DMA,
            pltpu.SemaphoreType.DMA,
            pltpu.SemaphoreType.DMA,
            pltpu.SemaphoreType.DMA,
            pltpu.SemaphoreType.DMA,
            pltpu.SemaphoreType.DMA,
        ],
    )
    return f(xl, xr, src, dst, att, zden)


_CH = 125
_NCH = _NGROUPS // _CH


def _sc_p2_body(xl_hbm, src_hbm, dst_hbm, w_hbm, den_hbm, zacc_hbm,
                accp_hbm,
                src_v, dst_v, xlr0, xlr1, wb0, wb1, denr0, denr1,
                msg0, msg1, acc_sh,
                semA0, semA1, semW0, semW1, semD0, semD1, semM0, semM1):
    c = lax.axis_index("c")
    s = lax.axis_index("s")
    wid = s * _NCORES + c
    gwbase = wid * _NGROUPS
    pltpu.sync_copy(zacc_hbm, acc_sh.at[pl.ds(s * _RPS, _RPS)])
    plsc.subcore_barrier()
    lanes = lax.iota(_i32, _G)

    def chunk(ci, carry0):
        cbase = ci * _CH

        def issue(g, xlr, wb, denr, semA, semW, semD):
            pltpu.async_copy(xl_hbm.at[src_v.at[g]], xlr, semA)
            pltpu.async_copy(w_hbm.at[gwbase + cbase + g], wb, semW)
            pltpu.async_copy(den_hbm.at[dst_v.at[g]], denr, semD)

        def wait_in(g, xlr, wb, denr, semA, semW, semD):
            pltpu.make_async_copy(xl_hbm.at[src_v.at[g]], xlr, semA).wait()
            pltpu.make_async_copy(
                w_hbm.at[gwbase + cbase + g], wb, semW).wait()
            pltpu.make_async_copy(den_hbm.at[dst_v.at[g]], denr, semD).wait()

        def compute(g, xlr, wb, denr, msg, semM):
            @pl.when(g >= 2)
            def _():
                pltpu.make_async_copy(
                    msg, acc_sh.at[dst_v.at[g - 2]], semM).wait()
            alpha_sc = []
            for h in range(_H):
                dh = plsc.load_gather(denr, [lanes, jnp.full((_G,), h, _i32)])
                ah = wb[h, :] * 0.125 / dh
                alpha_sc.append([ah[e] for e in range(_G)])

            def jloop(j, carry2):
                for e in range(_G):
                    mlo = jnp.zeros((16,), _f32)
                    mhi = jnp.zeros((16,), _f32)
                    for h in range(_H):
                        lo, hi = plsc.unpack(
                            xlr[e, pl.ds(h * _C + j * 32, 32)],
                            format=plsc.PackFormat.INTERLEAVED)
                        mlo = mlo + alpha_sc[h][e] * lo
                        mhi = mhi + alpha_sc[h][e] * hi
                    msg[e, pl.ds(j * 32, 16)] = mlo
                    msg[e, pl.ds(j * 32 + 16, 16)] = mhi
                return carry2

            lax.fori_loop(0, 4, jloop, 0)
            pltpu.async_copy(msg, acc_sh.at[dst_v.at[g]], semM, add=True)

        pltpu.sync_copy(src_hbm.at[wid, pl.ds(cbase, _CH)], src_v)
        pltpu.sync_copy(dst_hbm.at[wid, pl.ds(cbase, _CH)], dst_v)
        issue(0, xlr0, wb0, denr0, semA0, semW0, semD0)
        issue(1, xlr1, wb1, denr1, semA1, semW1, semD1)

        def pair(k, carry):
            a = k * 2
            b = a + 1
            wait_in(a, xlr0, wb0, denr0, semA0, semW0, semD0)
            compute(a, xlr0, wb0, denr0, msg0, semM0)

            @pl.when(a + 2 < _CH)
            def _():
                issue(a + 2, xlr0, wb0, denr0, semA0, semW0, semD0)

            @pl.when(b < _CH)
            def _():
                wait_in(b, xlr1, wb1, denr1, semA1, semW1, semD1)
                compute(b, xlr1, wb1, denr1, msg1, semM1)

                @pl.when(b + 2 < _CH)
                def _():
                    issue(b + 2, xlr1, wb1, denr1, semA1, semW1, semD1)
            return carry

        lax.fori_loop(0, (_CH + 1) // 2, pair, 0)
        pltpu.make_async_copy(msg0, acc_sh.at[dst_v.at[_CH - 1]], semM0).wait()
        pltpu.make_async_copy(msg1, acc_sh.at[dst_v.at[_CH - 2]], semM1).wait()
        return carry0

    lax.fori_loop(0, _NCH, chunk, 0)
    plsc.subcore_barrier()
    pltpu.sync_copy(acc_sh.at[pl.ds(s * _RPS, _RPS)],
                    accp_hbm.at[c, pl.ds(s * _RPS, _RPS)])


def _sc_p2(xl, src, dst, w, den, zacc):
    f = pl.kernel(
        _sc_p2_body,
        out_type=jax.ShapeDtypeStruct((_NCORES, _N, _C), _f32),
        mesh=_SC_MESH,
        compiler_params=_SC_PARAMS,
        scratch_types=[
            pltpu.VMEM((_CH, _G), _i32),
            pltpu.VMEM((_CH, _G), _i32),
            pltpu.VMEM((_G, _HC), _bf16),
            pltpu.VMEM((_G, _HC), _bf16),
            pltpu.VMEM((_H, _G), _f32),
            pltpu.VMEM((_H, _G), _f32),
            pltpu.VMEM((_G, 16), _f32),
            pltpu.VMEM((_G, 16), _f32),
            pltpu.VMEM((_G, _C), _f32),
            pltpu.VMEM((_G, _C), _f32),
            pltpu.VMEM_SHARED((_N, _C), _f32),
            pltpu.SemaphoreType.DMA,
            pltpu.SemaphoreType.DMA,
            pltpu.SemaphoreType.DMA,
            pltpu.SemaphoreType.DMA,
            pltpu.SemaphoreType.DMA,
            pltpu.SemaphoreType.DMA,
            pltpu.SemaphoreType.DMA,
            pltpu.SemaphoreType.DMA,
        ],
    )
    return f(xl, src, dst, w, den, zacc)


def _gat_layer(xv, src, dst, att, zden, zacc):
    xl, xr = xv
    w, den_parts = _sc_p1(xl, xr, src, dst, att.reshape(_HC), zden)
    den = _den_sum(den_parts.reshape(_NCORES, _N * 16))
    acc_parts = _sc_p2(xl, src, dst, w, den, zacc)
    return acc_parts


def kernel(x, edge_index, xyz, Wl1, Wr1, att1, b1, Wl2, Wr2, att2, b2,
           Wlin, blin):
    src = edge_index[0].reshape(_NW, _NGROUPS, _G)
    dst = edge_index[1].reshape(_NW, _NGROUPS, _G)
    zden = jnp.zeros((_RPS, 16), _f32)
    zacc = jnp.zeros((_RPS, _C), _f32)
    cp = jnp.asarray(_COLPERM)
    Wl1p, Wr1p = Wl1[:, cp], Wr1[:, cp]
    Wl2p, Wr2p = Wl2[:, cp], Wr2[:, cp]

    xl1, xr1, res1 = _proj_res(x, Wl1p, Wr1p, Wlin, blin)
    acc1 = _gat_layer((xl1, xr1), src, dst, att1, zden, zacc)
    xl2, xr2 = _mid(acc1, b1, Wl2p, Wr2p)
    acc2 = _gat_layer((xl2, xr2), src, dst, att2, zden, zacc)
    return _final(acc2, b2, res1)
